# baseline (speedup 1.0000x reference)
import jax
import jax.numpy as jnp
from jax.experimental import pallas as pl
from jax.experimental.pallas import tpu as pltpu

D_MODEL = 4096
NUM_EXPERTS = 64
BLOCK_M = 1024
N_CHUNK = 8
CHUNK_M = BLOCK_M // N_CHUNK


def _issue_block(x_hbm, xbuf, sems, block_idx, slot):
    for c in range(N_CHUNK):
        pltpu.make_async_copy(
            x_hbm.at[pl.ds(block_idx * BLOCK_M + c * CHUNK_M, CHUNK_M), :],
            xbuf.at[slot, pl.ds(c * CHUNK_M, CHUNK_M), :],
            sems.at[slot, c],
        ).start()


def _wait_block(x_hbm, xbuf, sems, slot):
    for c in range(N_CHUNK):
        pltpu.make_async_copy(
            x_hbm.at[pl.ds(c * CHUNK_M, CHUNK_M), :],
            xbuf.at[slot, pl.ds(c * CHUNK_M, CHUNK_M), :],
            sems.at[slot, c],
        ).wait()


def _router_kernel(idx_ref, x_hbm, w_ref, o_ref, xbuf, sems):
    del idx_ref
    i = pl.program_id(0)
    n = pl.num_programs(0)
    slot = jax.lax.rem(i, 2)
    nxt = jax.lax.rem(i + 1, 2)

    @pl.when(i == 0)
    def _prologue():
        _issue_block(x_hbm, xbuf, sems, 0, 0)

    @pl.when(i + 1 < n)
    def _prefetch_next():
        _issue_block(x_hbm, xbuf, sems, i + 1, nxt)

    _wait_block(x_hbm, xbuf, sems, slot)

    o_ref[...] = jax.lax.dot_general(
        xbuf[slot],
        w_ref[0],
        (((1,), (1,)), ((), ())),
        preferred_element_type=jnp.float32,
    )


def kernel(index, hidden_states, W):
    m = hidden_states.shape[0]
    idx = jnp.asarray(index, dtype=jnp.int32).reshape((1,))
    grid_spec = pltpu.PrefetchScalarGridSpec(
        num_scalar_prefetch=1,
        grid=(m // BLOCK_M,),
        in_specs=[
            pl.BlockSpec(memory_space=pl.ANY),
            pl.BlockSpec(
                (1, NUM_EXPERTS, D_MODEL), lambda i, idx_ref: (idx_ref[0], 0, 0)
            ),
        ],
        out_specs=pl.BlockSpec((BLOCK_M, NUM_EXPERTS), lambda i, idx_ref: (i, 0)),
        scratch_shapes=[
            pltpu.VMEM((2, BLOCK_M, D_MODEL), jnp.float32),
            pltpu.SemaphoreType.DMA((2, N_CHUNK)),
        ],
    )
    return pl.pallas_call(
        _router_kernel,
        grid_spec=grid_spec,
        out_shape=jax.ShapeDtypeStruct((m, NUM_EXPERTS), jnp.float32),
        compiler_params=pltpu.CompilerParams(
            dimension_semantics=("arbitrary",),
        ),
    )(idx, hidden_states, W)


# speedup vs baseline: 1.0260x; 1.0260x over previous
import jax
import jax.numpy as jnp
from jax.experimental import pallas as pl
from jax.experimental.pallas import tpu as pltpu

D_MODEL = 4096
NUM_EXPERTS = 64
BLOCK_M = 512


def _router_kernel(idx_ref, x_ref, w_hbm, o_ref, wbuf, wsem):
    @pl.when(pl.program_id(0) == 0)
    def _fetch_w():
        cp = pltpu.make_async_copy(w_hbm.at[idx_ref[0]], wbuf, wsem)
        cp.start()
        cp.wait()

    o_ref[...] = jax.lax.dot_general(
        x_ref[...],
        wbuf[...],
        (((1,), (1,)), ((), ())),
        preferred_element_type=jnp.float32,
    )


def kernel(index, hidden_states, W):
    m = hidden_states.shape[0]
    idx = jnp.asarray(index, dtype=jnp.int32).reshape((1,))
    grid_spec = pltpu.PrefetchScalarGridSpec(
        num_scalar_prefetch=1,
        grid=(m // BLOCK_M,),
        in_specs=[
            pl.BlockSpec((BLOCK_M, D_MODEL), lambda i, idx_ref: (i, 0)),
            pl.BlockSpec(memory_space=pl.ANY),
        ],
        out_specs=pl.BlockSpec((BLOCK_M, NUM_EXPERTS), lambda i, idx_ref: (i, 0)),
        scratch_shapes=[
            pltpu.VMEM((NUM_EXPERTS, D_MODEL), jnp.float32),
            pltpu.SemaphoreType.DMA,
        ],
    )
    return pl.pallas_call(
        _router_kernel,
        grid_spec=grid_spec,
        out_shape=jax.ShapeDtypeStruct((m, NUM_EXPERTS), jnp.float32),
    )(idx, hidden_states, W)


# speedup vs baseline: 1.0767x; 1.0495x over previous
import jax
import jax.numpy as jnp
from jax.experimental import pallas as pl
from jax.experimental.pallas import tpu as pltpu

D_MODEL = 4096
NUM_EXPERTS = 64
BLOCK_M = 512


def _router_kernel(idx_ref, x_ref, w_hbm, ot_ref, wbuf, wsem):
    @pl.when(pl.program_id(0) == 0)
    def _fetch_w():
        cp = pltpu.make_async_copy(w_hbm.at[idx_ref[0]], wbuf, wsem)
        cp.start()
        cp.wait()

    r = jax.lax.dot_general(
        x_ref[...],
        wbuf[...],
        (((1,), (1,)), ((), ())),
        preferred_element_type=jnp.float32,
    )
    ot_ref[...] = r.T


def kernel(index, hidden_states, W):
    m = hidden_states.shape[0]
    idx = jnp.asarray(index, dtype=jnp.int32).reshape((1,))
    grid_spec = pltpu.PrefetchScalarGridSpec(
        num_scalar_prefetch=1,
        grid=(m // BLOCK_M,),
        in_specs=[
            pl.BlockSpec((BLOCK_M, D_MODEL), lambda i, idx_ref: (i, 0)),
            pl.BlockSpec(memory_space=pl.ANY),
        ],
        out_specs=pl.BlockSpec((NUM_EXPERTS, BLOCK_M), lambda i, idx_ref: (0, i)),
        scratch_shapes=[
            pltpu.VMEM((NUM_EXPERTS, D_MODEL), jnp.float32),
            pltpu.SemaphoreType.DMA,
        ],
    )
    out_t = pl.pallas_call(
        _router_kernel,
        grid_spec=grid_spec,
        out_shape=jax.ShapeDtypeStruct((NUM_EXPERTS, m), jnp.float32),
    )(idx, hidden_states, W)
    return out_t.T
